# baseline (speedup 1.0000x reference)
import functools

import jax
import jax.numpy as jnp
from jax import lax
from jax.experimental import pallas as pl
from jax.experimental.pallas import tpu as pltpu
from jax.experimental.pallas import tpu_sc as plsc

N = 32768
D = 512
NUM_CLASSES = 3

NC, NS = 2, 16
NW = NC * NS
N_SC = 8192
N_TC = N - N_SC
BN = 4096
CH = 64
ROWS_PER_W = N_SC // NW
NCH = ROWS_PER_W // CH


def _sc_partial_kernel(x_hbm, p_hbm, o_hbm, xb0, xb1, pbuf, acc,
                       xs0, xs1):
    cid = lax.axis_index("c")
    sid = lax.axis_index("s")
    wid = sid * NC + cid
    wstart = N_TC + wid * ROWS_PER_W

    @pl.loop(0, NUM_CLASSES)
    def _(r):
        @pl.loop(0, D, step=16)
        def _(c1):
            acc.at[pl.ds(r, 1), pl.ds(c1, 16)][...] = jnp.zeros(
                (1, 16), jnp.float32)

    pltpu.sync_copy(p_hbm.at[pl.ds(wstart, ROWS_PER_W)], pbuf)

    xbufs = (xb0, xb1)
    xsems = (xs0, xs1)
    handles = [None] * NCH

    def issue(j):
        b = j % 2
        handles[j] = pltpu.async_copy(
            x_hbm.at[pl.ds(wstart + j * CH, CH)], xbufs[b], xsems[b])

    issue(0)
    for j in range(NCH):
        b = j % 2
        handles[j].wait()
        if j + 1 < NCH:
            issue(j + 1)
        buf = xbufs[b]

        @pl.loop(0, CH, step=16)
        def _(g):
            pv = pbuf[pl.ds(j * CH + g, 16)]
            for r16 in range(16):
                pr = pv[r16]

                @pl.loop(0, D, step=128)
                def _(c0):
                    for cu in range(0, 128, 16):
                        chunk = buf.at[pl.ds(g + r16, 1),
                                       pl.ds(c0 + cu, 16)][...]
                        plsc.addupdate(
                            acc.at[pl.ds(pr, 1), pl.ds(c0 + cu, 16)], chunk)

    pltpu.sync_copy(acc, o_hbm.at[wid])


def _sc_partial(x, p):
    mesh = plsc.VectorSubcoreMesh(core_axis_name="c", subcore_axis_name="s")
    return pl.kernel(
        _sc_partial_kernel,
        out_type=jax.ShapeDtypeStruct((NW, NUM_CLASSES, D), jnp.float32),
        mesh=mesh,
        scratch_types=[
            pltpu.VMEM((CH, D), jnp.float32),
            pltpu.VMEM((CH, D), jnp.float32),
            pltpu.VMEM((ROWS_PER_W,), jnp.int32),
            pltpu.VMEM((NUM_CLASSES, D), jnp.float32),
            pltpu.SemaphoreType.DMA,
            pltpu.SemaphoreType.DMA,
        ],
    )(x, p)


def _tc_sums_kernel(p_ref, x_ref, o_ref, acc_ref):
    i = pl.program_id(0)
    nsteps = pl.num_programs(0)

    @pl.when(i == 0)
    def _():
        acc_ref[...] = jnp.zeros_like(acc_ref)

    p = p_ref[0, :]
    x = x_ref[...]
    pb = jnp.broadcast_to(p[None, :], (8, BN))
    rows = lax.broadcasted_iota(jnp.int32, (8, BN), 0)
    onehot = (pb == rows).astype(jnp.float32)
    acc_ref[...] += lax.dot_general(
        onehot, x, (((1,), (0,)), ((), ())),
        preferred_element_type=jnp.float32,
        precision=lax.Precision.DEFAULT,
    )

    @pl.when(i == nsteps - 1)
    def _():
        o_ref[...] = acc_ref[0:NUM_CLASSES, :]


def _tc_sums(x, p2d):
    nsteps = N_TC // BN
    return pl.pallas_call(
        _tc_sums_kernel,
        grid=(nsteps,),
        in_specs=[
            pl.BlockSpec((1, BN), lambda i: (0, i)),
            pl.BlockSpec((BN, D), lambda i: (i, 0)),
        ],
        out_specs=pl.BlockSpec((NUM_CLASSES, D), lambda i: (0, 0)),
        out_shape=jax.ShapeDtypeStruct((NUM_CLASSES, D), jnp.float32),
        scratch_shapes=[pltpu.VMEM((8, D), jnp.float32)],
    )(p2d, x)


def _combine_kernel(p_ref, tc_ref, sc_ref, o_ref):
    p = p_ref[0, :]
    pb = jnp.broadcast_to(p[None, :], (8, N))
    rows = lax.broadcasted_iota(jnp.int32, (8, N), 0)
    counts = jnp.sum((pb == rows).astype(jnp.float32), axis=1,
                     keepdims=True)
    total = sc_ref[0]
    for w in range(1, NW):
        total = total + sc_ref[w]
    total = tc_ref[...] + total
    o_ref[...] = total / jnp.maximum(counts[0:NUM_CLASSES, :], 1.0)


def _combine(p2d, tc_sums, sc_partials):
    return pl.pallas_call(
        _combine_kernel,
        out_shape=jax.ShapeDtypeStruct((NUM_CLASSES, D), jnp.float32),
    )(p2d, tc_sums, sc_partials)


@jax.jit
def kernel(inputs, porality):
    p = porality.astype(jnp.int32)
    p2d = p.reshape(1, N)
    sc_partials = _sc_partial(inputs, p)
    tc_sums = _tc_sums(inputs, p2d)
    return _combine(p2d, tc_sums, sc_partials)


# speedup vs baseline: 1.4452x; 1.4452x over previous
import functools

import jax
import jax.numpy as jnp
from jax import lax
from jax.experimental import pallas as pl
from jax.experimental.pallas import tpu as pltpu
from jax.experimental.pallas import tpu_sc as plsc

N = 32768
D = 512
NUM_CLASSES = 3

NC, NS = 2, 16
NW = NC * NS
N_SC = 8192
N_TC = N - N_SC
BN = 4096
CH = 64
ROWS_PER_W = N_SC // NW
NCH = ROWS_PER_W // CH


def _sc_partial_kernel(x_hbm, p_hbm, o_hbm, xb0, xb1, pbuf, acc,
                       xs0, xs1):
    cid = lax.axis_index("c")
    sid = lax.axis_index("s")
    wid = sid * NC + cid
    wstart = N_TC + wid * ROWS_PER_W

    @pl.loop(0, NUM_CLASSES)
    def _(r):
        @pl.loop(0, D, step=16)
        def _(c1):
            acc.at[pl.ds(r, 1), pl.ds(c1, 16)][...] = jnp.zeros(
                (1, 16), jnp.float32)

    pltpu.sync_copy(p_hbm.at[pl.ds(wstart, ROWS_PER_W)], pbuf)

    xbufs = (xb0, xb1)
    xsems = (xs0, xs1)
    handles = [None] * NCH

    def issue(j):
        b = j % 2
        handles[j] = pltpu.async_copy(
            x_hbm.at[pl.ds(wstart + j * CH, CH)], xbufs[b], xsems[b])

    issue(0)
    for j in range(NCH):
        b = j % 2
        handles[j].wait()
        if j + 1 < NCH:
            issue(j + 1)
        buf = xbufs[b]

        @pl.loop(0, CH, step=16)
        def _(g):
            pv = pbuf[pl.ds(j * CH + g, 16)]
            for r16 in range(16):
                pr = pv[r16]
                row = buf.at[pl.ds(g + r16, 1), :][...]
                plsc.addupdate(acc.at[pl.ds(pr, 1), :], row)

    pltpu.sync_copy(acc, o_hbm.at[wid])


def _sc_partial(x, p):
    mesh = plsc.VectorSubcoreMesh(core_axis_name="c", subcore_axis_name="s")
    return pl.kernel(
        _sc_partial_kernel,
        out_type=jax.ShapeDtypeStruct((NW, NUM_CLASSES, D), jnp.float32),
        mesh=mesh,
        scratch_types=[
            pltpu.VMEM((CH, D), jnp.float32),
            pltpu.VMEM((CH, D), jnp.float32),
            pltpu.VMEM((ROWS_PER_W,), jnp.int32),
            pltpu.VMEM((NUM_CLASSES, D), jnp.float32),
            pltpu.SemaphoreType.DMA,
            pltpu.SemaphoreType.DMA,
        ],
    )(x, p)


def _tc_sums_kernel(p_ref, x_ref, o_ref, acc_ref):
    i = pl.program_id(0)
    nsteps = pl.num_programs(0)

    @pl.when(i == 0)
    def _():
        acc_ref[...] = jnp.zeros_like(acc_ref)

    p = p_ref[0, :]
    x = x_ref[...]
    pb = jnp.broadcast_to(p[None, :], (8, BN))
    rows = lax.broadcasted_iota(jnp.int32, (8, BN), 0)
    onehot = (pb == rows).astype(jnp.float32)
    acc_ref[...] += lax.dot_general(
        onehot, x, (((1,), (0,)), ((), ())),
        preferred_element_type=jnp.float32,
        precision=lax.Precision.DEFAULT,
    )

    @pl.when(i == nsteps - 1)
    def _():
        o_ref[...] = acc_ref[0:NUM_CLASSES, :]


def _tc_sums(x, p2d):
    nsteps = N_TC // BN
    return pl.pallas_call(
        _tc_sums_kernel,
        grid=(nsteps,),
        in_specs=[
            pl.BlockSpec((1, BN), lambda i: (0, i)),
            pl.BlockSpec((BN, D), lambda i: (i, 0)),
        ],
        out_specs=pl.BlockSpec((NUM_CLASSES, D), lambda i: (0, 0)),
        out_shape=jax.ShapeDtypeStruct((NUM_CLASSES, D), jnp.float32),
        scratch_shapes=[pltpu.VMEM((8, D), jnp.float32)],
    )(p2d, x)


def _combine_kernel(p_ref, tc_ref, sc_ref, o_ref):
    p = p_ref[0, :]
    pb = jnp.broadcast_to(p[None, :], (8, N))
    rows = lax.broadcasted_iota(jnp.int32, (8, N), 0)
    counts = jnp.sum((pb == rows).astype(jnp.float32), axis=1,
                     keepdims=True)
    total = sc_ref[0]
    for w in range(1, NW):
        total = total + sc_ref[w]
    total = tc_ref[...] + total
    o_ref[...] = total / jnp.maximum(counts[0:NUM_CLASSES, :], 1.0)


def _combine(p2d, tc_sums, sc_partials):
    return pl.pallas_call(
        _combine_kernel,
        out_shape=jax.ShapeDtypeStruct((NUM_CLASSES, D), jnp.float32),
    )(p2d, tc_sums, sc_partials)


@jax.jit
def kernel(inputs, porality):
    p = porality.astype(jnp.int32)
    p2d = p.reshape(1, N)
    sc_partials = _sc_partial(inputs, p)
    tc_sums = _tc_sums(inputs, p2d)
    return _combine(p2d, tc_sums, sc_partials)


# speedup vs baseline: 1.6846x; 1.1657x over previous
import functools

import jax
import jax.numpy as jnp
from jax import lax
from jax.experimental import pallas as pl
from jax.experimental.pallas import tpu as pltpu
from jax.experimental.pallas import tpu_sc as plsc

N = 32768
D = 512
NUM_CLASSES = 3

NC, NS = 2, 16
NW = NC * NS
N_SC = 2048
N_TC = N - N_SC
BN = 3840
CH = 64
ROWS_PER_W = N_SC // NW
NCH = ROWS_PER_W // CH


def _sc_partial_kernel(x_hbm, p_hbm, o_hbm, xb0, xb1, pbuf, acc,
                       xs0, xs1):
    cid = lax.axis_index("c")
    sid = lax.axis_index("s")
    wid = sid * NC + cid
    wstart = N_TC + wid * ROWS_PER_W

    @pl.loop(0, NUM_CLASSES)
    def _(r):
        @pl.loop(0, D, step=16)
        def _(c1):
            acc.at[pl.ds(r, 1), pl.ds(c1, 16)][...] = jnp.zeros(
                (1, 16), jnp.float32)

    pltpu.sync_copy(p_hbm.at[pl.ds(wstart, ROWS_PER_W)], pbuf)

    xbufs = (xb0, xb1)
    xsems = (xs0, xs1)
    handles = [None] * NCH

    def issue(j):
        b = j % 2
        handles[j] = pltpu.async_copy(
            x_hbm.at[pl.ds(wstart + j * CH, CH)], xbufs[b], xsems[b])

    issue(0)
    for j in range(NCH):
        b = j % 2
        handles[j].wait()
        if j + 1 < NCH:
            issue(j + 1)
        buf = xbufs[b]

        @pl.loop(0, CH, step=16)
        def _(g):
            pv = pbuf[pl.ds(j * CH + g, 16)]
            for r16 in range(16):
                pr = pv[r16]
                row = buf.at[pl.ds(g + r16, 1), :][...]
                plsc.addupdate(acc.at[pl.ds(pr, 1), :], row)

    pltpu.sync_copy(acc, o_hbm.at[wid])


def _sc_partial(x, p):
    mesh = plsc.VectorSubcoreMesh(core_axis_name="c", subcore_axis_name="s")
    return pl.kernel(
        _sc_partial_kernel,
        out_type=jax.ShapeDtypeStruct((NW, NUM_CLASSES, D), jnp.float32),
        mesh=mesh,
        scratch_types=[
            pltpu.VMEM((CH, D), jnp.float32),
            pltpu.VMEM((CH, D), jnp.float32),
            pltpu.VMEM((ROWS_PER_W,), jnp.int32),
            pltpu.VMEM((NUM_CLASSES, D), jnp.float32),
            pltpu.SemaphoreType.DMA,
            pltpu.SemaphoreType.DMA,
        ],
    )(x, p)


def _tc_sums_kernel(p_ref, x_ref, o_ref, acc_ref):
    i = pl.program_id(0)
    nsteps = pl.num_programs(0)

    @pl.when(i == 0)
    def _():
        acc_ref[...] = jnp.zeros_like(acc_ref)

    p = p_ref[0, :]
    x = x_ref[...]
    pb = jnp.broadcast_to(p[None, :], (8, BN))
    rows = lax.broadcasted_iota(jnp.int32, (8, BN), 0)
    onehot = (pb == rows).astype(jnp.float32)
    acc_ref[...] += lax.dot_general(
        onehot, x, (((1,), (0,)), ((), ())),
        preferred_element_type=jnp.float32,
        precision=lax.Precision.DEFAULT,
    )

    @pl.when(i == nsteps - 1)
    def _():
        o_ref[...] = acc_ref[0:NUM_CLASSES, :]


def _tc_sums(x, p2d):
    nsteps = N_TC // BN
    return pl.pallas_call(
        _tc_sums_kernel,
        grid=(nsteps,),
        in_specs=[
            pl.BlockSpec((1, BN), lambda i: (0, i)),
            pl.BlockSpec((BN, D), lambda i: (i, 0)),
        ],
        out_specs=pl.BlockSpec((NUM_CLASSES, D), lambda i: (0, 0)),
        out_shape=jax.ShapeDtypeStruct((NUM_CLASSES, D), jnp.float32),
        scratch_shapes=[pltpu.VMEM((8, D), jnp.float32)],
    )(p2d, x)


def _combine_kernel(p_ref, tc_ref, sc_ref, o_ref):
    p = p_ref[0, :]
    pb = jnp.broadcast_to(p[None, :], (8, N))
    rows = lax.broadcasted_iota(jnp.int32, (8, N), 0)
    counts = jnp.sum((pb == rows).astype(jnp.float32), axis=1,
                     keepdims=True)
    total = sc_ref[0]
    for w in range(1, NW):
        total = total + sc_ref[w]
    total = tc_ref[...] + total
    o_ref[...] = total / jnp.maximum(counts[0:NUM_CLASSES, :], 1.0)


def _combine(p2d, tc_sums, sc_partials):
    return pl.pallas_call(
        _combine_kernel,
        out_shape=jax.ShapeDtypeStruct((NUM_CLASSES, D), jnp.float32),
    )(p2d, tc_sums, sc_partials)


@jax.jit
def kernel(inputs, porality):
    p = porality.astype(jnp.int32)
    p2d = p.reshape(1, N)
    sc_partials = _sc_partial(inputs, p)
    tc_sums = _tc_sums(inputs, p2d)
    return _combine(p2d, tc_sums, sc_partials)
